# baseline (speedup 1.0000x reference)
import functools

import jax
import jax.numpy as jnp
from jax import lax
from jax.experimental import pallas as pl
from jax.experimental.pallas import tpu as pltpu
from jax.experimental.pallas import tpu_sc as plsc

def _lane_broadcast(v16, tvec):
    return lax.gather(
        v16, tvec[:, None],
        lax.GatherDimensionNumbers(
            offset_dims=(), collapsed_slice_dims=(0,), start_index_map=(0,)),
        (1,), mode=lax.GatherScatterMode.PROMISE_IN_BOUNDS)


def _lane_sum_splat(v16):
    idx = lax.iota(jnp.int32, 16)
    for sh in (8, 4, 2, 1):
        v16 = v16 + _lane_broadcast(v16, idx ^ sh)
    return v16


B, S = 4, 2048
HIDDEN = 768
EPS = 1e-12

NC, NS, L = 2, 16, 16
NW = NC * NS
N_TOK = B * S
TOK_PER_W = N_TOK // NW
CHUNK = 64
N_CHUNK = TOK_PER_W // CHUNK
NJ = HIDDEN // L


def _sc_body(ids_hbm, ttf_hbm, word_hbm, pos_hbm, type_hbm, gam_hbm, bet_hbm,
             out_hbm, idx_v, ttf_v, rows_v, pos_v, type_v, dt_v, gam_v, bet_v,
             sem):
    cid = lax.axis_index("c")
    sid = lax.axis_index("s")
    wid = cid * NS + sid

    pltpu.sync_copy(type_hbm, type_v)
    pltpu.sync_copy(gam_hbm, gam_v)
    pltpu.sync_copy(bet_hbm, bet_v)
    for j in range(NJ):
        sl = pl.ds(j * L, L)
        dt_v[sl] = type_v[1, sl] - type_v[0, sl]

    def chunk_body(c, _):
        base = wid * TOK_PER_W + c * CHUNK
        s0 = lax.rem(base, S)
        pltpu.sync_copy(ids_hbm.at[pl.ds(base, CHUNK)], idx_v)
        pltpu.sync_copy(ttf_hbm.at[pl.ds(base, CHUNK)], ttf_v)
        pltpu.sync_copy(pos_hbm.at[pl.ds(s0, CHUNK)], pos_v)
        pltpu.async_copy(word_hbm.at[idx_v], rows_v, sem).wait()

        def tok_body(t, _):
            g = t // L
            tt16 = ttf_v[pl.ds(g * L, L)]
            tvec = jnp.full((L,), lax.rem(t, L), dtype=jnp.int32)
            ttk = _lane_broadcast(tt16, tvec)
            acc = jnp.zeros((L,), jnp.float32)
            acc2 = jnp.zeros((L,), jnp.float32)
            for j in range(NJ):
                sl = pl.ds(j * L, L)
                v = rows_v[t, sl] + pos_v[t, sl] + type_v[0, sl] + ttk * dt_v[sl]
                rows_v[t, sl] = v
                acc = acc + v
                acc2 = acc2 + v * v
            meanv = _lane_sum_splat(acc) * (1.0 / HIDDEN)
            m2v = _lane_sum_splat(acc2) * (1.0 / HIDDEN)
            varv = m2v - meanv * meanv + EPS
            iv = plsc.bitcast(varv, jnp.int32)
            iv = 0x5F3759DF - (iv >> 1)
            y = plsc.bitcast(iv, jnp.float32)
            for _ in range(4):
                y = y * (1.5 - 0.5 * varv * y * y)
            for j in range(NJ):
                sl = pl.ds(j * L, L)
                rows_v[t, sl] = (rows_v[t, sl] - meanv) * y * gam_v[sl] + bet_v[sl]
            return 0

        lax.fori_loop(0, CHUNK, tok_body, 0)
        pltpu.sync_copy(rows_v, out_hbm.at[pl.ds(base, CHUNK)])
        return 0

    lax.fori_loop(0, N_CHUNK, chunk_body, 0)


@jax.jit
def _run(ids, ttf, word_emb, pos_emb, type_emb, gamma, beta):
    mesh = plsc.VectorSubcoreMesh(core_axis_name="c", subcore_axis_name="s",
                                  num_cores=NC, num_subcores=NS)
    f = pl.kernel(
        _sc_body,
        out_type=jax.ShapeDtypeStruct((N_TOK, HIDDEN), jnp.float32),
        mesh=mesh,
        compiler_params=pltpu.CompilerParams(needs_layout_passes=False),
        scratch_types=[
            pltpu.VMEM((CHUNK,), jnp.int32),
            pltpu.VMEM((CHUNK,), jnp.float32),
            pltpu.VMEM((CHUNK, HIDDEN), jnp.float32),
            pltpu.VMEM((CHUNK, HIDDEN), jnp.float32),
            pltpu.VMEM((2, HIDDEN), jnp.float32),
            pltpu.VMEM((HIDDEN,), jnp.float32),
            pltpu.VMEM((HIDDEN,), jnp.float32),
            pltpu.VMEM((HIDDEN,), jnp.float32),
            pltpu.SemaphoreType.DMA,
        ],
    )
    return f(ids, ttf, word_emb, pos_emb, type_emb, gamma, beta)


def kernel(input_ids, token_type_ids, word_emb, pos_emb, type_emb, gamma, beta):
    ids = input_ids.reshape(-1).astype(jnp.int32)
    ttf = token_type_ids.reshape(-1).astype(jnp.float32)
    out = _run(ids, ttf, word_emb, pos_emb, type_emb, gamma, beta)
    return out.reshape(B, S, HIDDEN)


# speedup vs baseline: 3.1601x; 3.1601x over previous
import jax
import jax.numpy as jnp
from jax import lax
from jax.experimental import pallas as pl
from jax.experimental.pallas import tpu as pltpu
from jax.experimental.pallas import tpu_sc as plsc

B, S = 4, 2048
HIDDEN = 768
EPS = 1e-12

NC, NS = 2, 16
NW = NC * NS
N_TOK = B * S
TOK_PER_W = N_TOK // NW
CHUNK = 64
N_CHUNK = TOK_PER_W // CHUNK

TC_BLK = 1024
N_BLK = N_TOK // TC_BLK


def _sc_gather_body(ids_hbm, word_hbm, out_hbm,
                    idx0, idx1, rows0, rows1, gs0, gs1, os0, os1):
    wid = lax.axis_index("c") * NS + lax.axis_index("s")
    base = wid * TOK_PER_W
    idx = (idx0, idx1)
    rows = (rows0, rows1)
    gsem = (gs0, gs1)
    osem = (os0, os1)

    out_copies = [None, None]
    pltpu.sync_copy(ids_hbm.at[pl.ds(base, CHUNK)], idx0)
    cur = pltpu.async_copy(word_hbm.at[idx0], rows0, gs0)
    for c in range(N_CHUNK):
        p = c & 1
        q = p ^ 1
        if c + 1 < N_CHUNK:
            if out_copies[q] is not None:
                out_copies[q].wait()
            pltpu.sync_copy(ids_hbm.at[pl.ds(base + (c + 1) * CHUNK, CHUNK)],
                            idx[q])
            nxt = pltpu.async_copy(word_hbm.at[idx[q]], rows[q], gsem[q])
        cur.wait()
        out_copies[p] = pltpu.async_copy(
            rows[p], out_hbm.at[pl.ds(base + c * CHUNK, CHUNK)], osem[p])
        if c + 1 < N_CHUNK:
            cur = nxt
    out_copies[0].wait()
    out_copies[1].wait()


def _sc_gather(ids, word_emb):
    mesh = plsc.VectorSubcoreMesh(core_axis_name="c", subcore_axis_name="s",
                                  num_cores=NC, num_subcores=NS)
    f = pl.kernel(
        _sc_gather_body,
        out_type=jax.ShapeDtypeStruct((N_TOK, HIDDEN), jnp.float32),
        mesh=mesh,
        compiler_params=pltpu.CompilerParams(needs_layout_passes=False),
        scratch_types=[
            pltpu.VMEM((CHUNK,), jnp.int32),
            pltpu.VMEM((CHUNK,), jnp.int32),
            pltpu.VMEM((CHUNK, HIDDEN), jnp.float32),
            pltpu.VMEM((CHUNK, HIDDEN), jnp.float32),
            pltpu.SemaphoreType.DMA,
            pltpu.SemaphoreType.DMA,
            pltpu.SemaphoreType.DMA,
            pltpu.SemaphoreType.DMA,
        ],
    )
    return f(ids, word_emb)


def _tc_body(gath_ref, pos_ref, ttf_ref, type_ref, gam_ref, bet_ref, out_ref):
    x = gath_ref[...] + pos_ref[...]
    t0 = type_ref[0:1, :]
    dt = type_ref[1:2, :] - t0
    x = x + t0 + ttf_ref[...] * dt
    mean = jnp.mean(x, axis=-1, keepdims=True)
    cent = x - mean
    var = jnp.mean(cent * cent, axis=-1, keepdims=True)
    normed = cent * lax.rsqrt(var + EPS)
    out_ref[...] = normed * gam_ref[...] + bet_ref[...]


def _tc_ln(gathered, ttf2d, pos_emb, type_emb, gamma, beta):
    return pl.pallas_call(
        _tc_body,
        grid=(N_BLK,),
        in_specs=[
            pl.BlockSpec((TC_BLK, HIDDEN), lambda g: (g, 0)),
            pl.BlockSpec((TC_BLK, HIDDEN), lambda g: (g % (S // TC_BLK), 0)),
            pl.BlockSpec((TC_BLK, 1), lambda g: (g, 0)),
            pl.BlockSpec((2, HIDDEN), lambda g: (0, 0)),
            pl.BlockSpec((1, HIDDEN), lambda g: (0, 0)),
            pl.BlockSpec((1, HIDDEN), lambda g: (0, 0)),
        ],
        out_specs=pl.BlockSpec((TC_BLK, HIDDEN), lambda g: (g, 0)),
        out_shape=jax.ShapeDtypeStruct((N_TOK, HIDDEN), jnp.float32),
    )(gathered, pos_emb, ttf2d, type_emb, gamma, beta)


@jax.jit
def _run(ids, ttf2d, word_emb, pos_emb, type_emb, gamma2d, beta2d):
    gathered = _sc_gather(ids, word_emb)
    return _tc_ln(gathered, ttf2d, pos_emb, type_emb, gamma2d, beta2d)


def kernel(input_ids, token_type_ids, word_emb, pos_emb, type_emb, gamma, beta):
    ids = input_ids.reshape(-1).astype(jnp.int32)
    ttf2d = token_type_ids.reshape(-1, 1).astype(jnp.float32)
    out = _run(ids, ttf2d, word_emb, pos_emb, type_emb,
               gamma.reshape(1, HIDDEN), beta.reshape(1, HIDDEN))
    return out.reshape(B, S, HIDDEN)
